# baseline (speedup 1.0000x reference)
import jax
import jax.numpy as jnp
from jax.experimental import pallas as pl

NBINS = 100
ROWS_PER_BLOCK = 8


def _norm_body(t_ref, mean_ref, std_ref, x_ref, o_ref):
    tb = t_ref[...]
    bins = jnp.clip((tb * NBINS).astype(jnp.int32), 0, NBINS - 1)
    lanes = jax.lax.broadcasted_iota(jnp.int32, (1, 128), 1)
    oh = bins == lanes
    m = jnp.sum(jnp.where(oh, mean_ref[...], 0.0), axis=1, keepdims=True)
    s = jnp.sum(jnp.where(oh, std_ref[...], 0.0), axis=1, keepdims=True)
    inv = 1.0 / s
    o_ref[...] = (x_ref[...] - m) * inv


def kernel(x_t, t, data_mean, data_std):
    B = x_t.shape[0]
    F = x_t.size // B
    x2 = x_t.reshape(B, F)
    t2 = t.reshape(B, 1)
    mean_p = jnp.zeros((1, 128), jnp.float32).at[0, :NBINS].set(data_mean)
    std_p = jnp.ones((1, 128), jnp.float32).at[0, :NBINS].set(data_std)

    R = ROWS_PER_BLOCK
    grid = (B // R,)
    out = pl.pallas_call(
        _norm_body,
        grid=grid,
        in_specs=[
            pl.BlockSpec((R, 1), lambda i: (i, 0)),
            pl.BlockSpec((1, 128), lambda i: (0, 0)),
            pl.BlockSpec((1, 128), lambda i: (0, 0)),
            pl.BlockSpec((R, F), lambda i: (i, 0)),
        ],
        out_specs=pl.BlockSpec((R, F), lambda i: (i, 0)),
        out_shape=jax.ShapeDtypeStruct((B, F), jnp.float32),
    )(t2, mean_p, std_p, x2)
    return out.reshape(x_t.shape)
